# baseline (speedup 1.0000x reference)
import math

import jax
import jax.numpy as jnp
from jax import lax
from jax.experimental import pallas as pl
from jax.experimental.pallas import tpu as pltpu
from jax.experimental.pallas import tpu_sc as plsc

N = 10000
E = 320000
L = 4
H = 128
C = 40
ALPHA = 0.1
LAMBDA = 0.5
BN_EPS = 1e-5

NC = 2
NS = 16
K = 128
NCHUNK = E // K
CHUNK_PER_SC = NCHUNK // NC
ROWS_PER_TILE = CHUNK_PER_SC // NS
LEFTOVER = CHUNK_PER_SC - ROWS_PER_TILE * NS
NODE_PER_TILE = N // NS
IB = 26

K2 = 64
NCH2 = E // K2
TPT = 160
NCH2P = NC * NS * TPT
IB2 = 20
NB = 4
AHEAD = 3


def _sc_mesh():
    return plsc.VectorSubcoreMesh(core_axis_name="c", subcore_axis_name="s")


def _hist_body(dst3_ref, ones_ref, zvec_ref, out_ref, idxb, ones_v,
               acc0, acc1, acc2, acc3):
    c = lax.axis_index("c")
    s = lax.axis_index("s")
    accs = [acc0, acc1, acc2, acc3]

    @pl.when(s == 0)
    def _():
        for a in accs:
            pltpu.sync_copy(zvec_ref, a)

    pltpu.sync_copy(ones_ref, ones_v)
    plsc.subcore_barrier()

    row0 = c * CHUNK_PER_SC + s * ROWS_PER_TILE
    for l in range(L):
        pltpu.sync_copy(dst3_ref.at[l, pl.ds(row0, ROWS_PER_TILE), :], idxb)

        def body(j, carry, l=l):
            pltpu.sync_copy(ones_v, accs[l].at[idxb.at[j]], add=True)
            return carry

        lax.fori_loop(0, ROWS_PER_TILE, body, 0)

        @pl.when(s < LEFTOVER)
        def _(l=l):
            rowx = c * CHUNK_PER_SC + NS * ROWS_PER_TILE + s
            pltpu.sync_copy(dst3_ref.at[l, rowx, :], idxb.at[0])
            pltpu.sync_copy(ones_v, accs[l].at[idxb.at[0]], add=True)

    plsc.subcore_barrier()

    @pl.when(s == 0)
    def _():
        for l in range(L):
            pltpu.sync_copy(accs[l], out_ref.at[c, l])


def _hist(dst3, ones_k, zvec):
    f = pl.kernel(
        _hist_body,
        out_type=jax.ShapeDtypeStruct((NC, L, N), jnp.float32),
        mesh=_sc_mesh(),
        scratch_types=[
            pltpu.VMEM((ROWS_PER_TILE, K), jnp.int32),
            pltpu.VMEM((K,), jnp.float32),
            pltpu.VMEM_SHARED((N,), jnp.float32),
            pltpu.VMEM_SHARED((N,), jnp.float32),
            pltpu.VMEM_SHARED((N,), jnp.float32),
            pltpu.VMEM_SHARED((N,), jnp.float32),
        ],
        compiler_params=pltpu.CompilerParams(use_tc_tiling_on_sc=False),
        name="gcnii_degree_hist",
    )
    return f(dst3, ones_k, zvec)


def _edge_body(src2_ref, dst2_ref, hn_ref, zrows_ref, out_ref,
               idxsb, idxdb, rb0, rb1, rb2, rb3, acc,
               sem0, sem1, sem2, sem3):
    bufs = [rb0, rb1, rb2, rb3]
    sems = [sem0, sem1, sem2, sem3]
    c = lax.axis_index("c")
    s = lax.axis_index("s")
    r0 = s * NODE_PER_TILE
    pltpu.sync_copy(zrows_ref.at[pl.ds(r0, NODE_PER_TILE), :],
                    acc.at[pl.ds(r0, NODE_PER_TILE), :])
    plsc.subcore_barrier()

    base = (c * NS + s) * TPT

    def blk_body(blk, carry):
        b0 = base + blk * IB2
        pltpu.sync_copy(src2_ref.at[pl.ds(b0, IB2), :], idxsb)
        pltpu.sync_copy(dst2_ref.at[pl.ds(b0, IB2), :], idxdb)

        for t in range(AHEAD):
            pltpu.async_copy(hn_ref.at[idxsb.at[t]], bufs[t], sems[t])
        for i in range(IB2):
            b = i % NB
            pltpu.make_async_copy(hn_ref.at[idxsb.at[i]], bufs[b],
                                  sems[b]).wait()
            if i + AHEAD < IB2:
                bn = (i + AHEAD) % NB
                pltpu.async_copy(hn_ref.at[idxsb.at[i + AHEAD]], bufs[bn],
                                 sems[bn])

            @pl.when(b0 + i < NCH2)
            def _(b=b, i=i):
                pltpu.sync_copy(bufs[b], acc.at[idxdb.at[i]], add=True)

        return carry

    lax.fori_loop(0, TPT // IB2, blk_body, 0)

    plsc.subcore_barrier()
    pltpu.sync_copy(acc.at[pl.ds(r0, NODE_PER_TILE), :],
                    out_ref.at[c, pl.ds(r0, NODE_PER_TILE), :])


def _edge_scatter(src2, dst2, hn, zrows):
    f = pl.kernel(
        _edge_body,
        out_type=jax.ShapeDtypeStruct((NC, N, H), jnp.float32),
        mesh=_sc_mesh(),
        scratch_types=[
            pltpu.VMEM((IB2, K2), jnp.int32),
            pltpu.VMEM((IB2, K2), jnp.int32),
            pltpu.VMEM((K2, H), jnp.float32),
            pltpu.VMEM((K2, H), jnp.float32),
            pltpu.VMEM((K2, H), jnp.float32),
            pltpu.VMEM((K2, H), jnp.float32),
            pltpu.VMEM_SHARED((N, H), jnp.float32),
            pltpu.SemaphoreType.DMA,
            pltpu.SemaphoreType.DMA,
            pltpu.SemaphoreType.DMA,
            pltpu.SemaphoreType.DMA,
        ],
        compiler_params=pltpu.CompilerParams(use_tc_tiling_on_sc=False),
        name="gcnii_edge_scatter",
    )
    return f(src2, dst2, hn, zrows)


def _tc_prep(x, W_in, b_in, cntT):
    def body(x_ref, wi_ref, bi_ref, cnt_ref, h_ref, hn0_ref, rinv_ref,
             dinv_ref):
        cnt = cnt_ref[0] + cnt_ref[1]
        deg = cnt + 1.0
        rinv = lax.rsqrt(deg)
        dinv = 1.0 / deg
        h = jnp.dot(x_ref[...], wi_ref[...],
                    preferred_element_type=jnp.float32) + bi_ref[...][None, :]
        h = jnp.maximum(h, 0.0)
        h_ref[...] = h
        hn0_ref[...] = h * rinv[:, 0:1]
        rinv_ref[...] = rinv
        dinv_ref[...] = dinv

    return pl.pallas_call(
        body,
        out_shape=[
            jax.ShapeDtypeStruct((N, H), jnp.float32),
            jax.ShapeDtypeStruct((N, H), jnp.float32),
            jax.ShapeDtypeStruct((N, L), jnp.float32),
            jax.ShapeDtypeStruct((N, L), jnp.float32),
        ],
        name="gcnii_prep",
    )(x, W_in, b_in, cntT)


def _tc_layer(l, beta, s_part, h, h0, rinv, dinv, W_l, gamma_l, betabn_l):
    def body(sp_ref, h_ref, h0_ref, rinv_ref, dinv_ref, w_ref, g_ref, bb_ref,
             hout_ref, hnout_ref):
        s = sp_ref[0] + sp_ref[1]
        agg = s * rinv_ref[:, l:l + 1] + h_ref[...] * dinv_ref[:, l:l + 1]
        sup = (1.0 - ALPHA) * agg + ALPHA * h0_ref[...]
        t = (1.0 - beta) * sup + beta * jnp.dot(
            sup, w_ref[...], preferred_element_type=jnp.float32)
        mean = jnp.mean(t, axis=0, keepdims=True)
        var = jnp.mean((t - mean) ** 2, axis=0, keepdims=True)
        hb = (t - mean) * lax.rsqrt(var + BN_EPS) * g_ref[...][None, :] \
            + bb_ref[...][None, :]
        hnew = jnp.maximum(hb, 0.0)
        hout_ref[...] = hnew
        hnout_ref[...] = hnew * rinv_ref[:, l + 1:l + 2]

    return pl.pallas_call(
        body,
        out_shape=[
            jax.ShapeDtypeStruct((N, H), jnp.float32),
            jax.ShapeDtypeStruct((N, H), jnp.float32),
        ],
        name=f"gcnii_layer{l}",
    )(s_part, h, h0, rinv, dinv, W_l, gamma_l, betabn_l)


def _tc_last(l, beta, s_part, h, h0, rinv, dinv, W_l, gamma_l, betabn_l,
             W_out, b_out):
    def body(sp_ref, h_ref, h0_ref, rinv_ref, dinv_ref, w_ref, g_ref, bb_ref,
             wo_ref, bo_ref, out_ref):
        s = sp_ref[0] + sp_ref[1]
        agg = s * rinv_ref[:, l:l + 1] + h_ref[...] * dinv_ref[:, l:l + 1]
        sup = (1.0 - ALPHA) * agg + ALPHA * h0_ref[...]
        t = (1.0 - beta) * sup + beta * jnp.dot(
            sup, w_ref[...], preferred_element_type=jnp.float32)
        mean = jnp.mean(t, axis=0, keepdims=True)
        var = jnp.mean((t - mean) ** 2, axis=0, keepdims=True)
        hb = (t - mean) * lax.rsqrt(var + BN_EPS) * g_ref[...][None, :] \
            + bb_ref[...][None, :]
        hnew = jnp.maximum(hb, 0.0)
        out_ref[...] = jnp.dot(hnew, wo_ref[...],
                               preferred_element_type=jnp.float32) \
            + bo_ref[...][None, :]

    return pl.pallas_call(
        body,
        out_shape=jax.ShapeDtypeStruct((N, C), jnp.float32),
        name="gcnii_last",
    )(s_part, h, h0, rinv, dinv, W_l, gamma_l, betabn_l, W_out, b_out)


def kernel(x, W_in, b_in, Ws, gammas, betas_bn, W_out, b_out, nodeblocks):
    dst3 = nodeblocks[:, 1, :].reshape(L, NCHUNK, K)
    ones_k = jnp.ones((K,), jnp.float32)
    zvec = jnp.zeros((N,), jnp.float32)
    zrows = jnp.zeros((N, H), jnp.float32)

    cnt_part = _hist(dst3, ones_k, zvec)
    cntT = jnp.transpose(cnt_part, (0, 2, 1))
    h, hn, rinv, dinv = _tc_prep(x, W_in, b_in, cntT)
    h0 = h
    out = None
    pad = jnp.zeros((NCH2P - NCH2, K2), jnp.int32)
    for l in range(L):
        src2 = jnp.concatenate(
            [nodeblocks[l, 0].reshape(NCH2, K2), pad], axis=0)
        dst2 = jnp.concatenate(
            [nodeblocks[l, 1].reshape(NCH2, K2), pad], axis=0)
        s_part = _edge_scatter(src2, dst2, hn, zrows)
        beta = float(math.log(LAMBDA / (l + 1) + 1.0))
        if l < L - 1:
            h, hn = _tc_layer(l, beta, s_part, h, h0, rinv, dinv,
                              Ws[l], gammas[l], betas_bn[l])
        else:
            out = _tc_last(l, beta, s_part, h, h0, rinv, dinv,
                           Ws[l], gammas[l], betas_bn[l], W_out, b_out)
    return out


# speedup vs baseline: 3.0958x; 3.0958x over previous
import math

import jax
import jax.numpy as jnp
from jax import lax
from jax.experimental import pallas as pl
from jax.experimental.pallas import tpu as pltpu
from jax.experimental.pallas import tpu_sc as plsc

N = 10000
E = 320000
L = 4
H = 128
C = 40
ALPHA = 0.1
LAMBDA = 0.5
BN_EPS = 1e-5

NC = 2
NS = 16
K = 128
NCHUNK = E // K
CHUNK_PER_SC = NCHUNK // NC
ROWS_PER_TILE = CHUNK_PER_SC // NS
LEFTOVER = CHUNK_PER_SC - ROWS_PER_TILE * NS
NODE_PER_TILE = N // NS
IB = 26


def _sc_mesh():
    return plsc.VectorSubcoreMesh(core_axis_name="c", subcore_axis_name="s")


def _hist_body(dst3_ref, ones_ref, zvec_ref, out_ref, idxb, ones_v,
               acc0, acc1, acc2, acc3):
    c = lax.axis_index("c")
    s = lax.axis_index("s")
    accs = [acc0, acc1, acc2, acc3]

    @pl.when(s == 0)
    def _():
        for a in accs:
            pltpu.sync_copy(zvec_ref, a)

    pltpu.sync_copy(ones_ref, ones_v)
    plsc.subcore_barrier()

    row0 = c * CHUNK_PER_SC + s * ROWS_PER_TILE
    for l in range(L):
        pltpu.sync_copy(dst3_ref.at[l, pl.ds(row0, ROWS_PER_TILE), :], idxb)

        def body(j, carry, l=l):
            pltpu.sync_copy(ones_v, accs[l].at[idxb.at[j]], add=True)
            return carry

        lax.fori_loop(0, ROWS_PER_TILE, body, 0)

        @pl.when(s < LEFTOVER)
        def _(l=l):
            rowx = c * CHUNK_PER_SC + NS * ROWS_PER_TILE + s
            pltpu.sync_copy(dst3_ref.at[l, rowx, :], idxb.at[0])
            pltpu.sync_copy(ones_v, accs[l].at[idxb.at[0]], add=True)

    plsc.subcore_barrier()

    @pl.when(s == 0)
    def _():
        for l in range(L):
            pltpu.sync_copy(accs[l], out_ref.at[c, l])


def _hist(dst3, ones_k, zvec):
    f = pl.kernel(
        _hist_body,
        out_type=jax.ShapeDtypeStruct((NC, L, N), jnp.float32),
        mesh=_sc_mesh(),
        scratch_types=[
            pltpu.VMEM((ROWS_PER_TILE, K), jnp.int32),
            pltpu.VMEM((K,), jnp.float32),
            pltpu.VMEM_SHARED((N,), jnp.float32),
            pltpu.VMEM_SHARED((N,), jnp.float32),
            pltpu.VMEM_SHARED((N,), jnp.float32),
            pltpu.VMEM_SHARED((N,), jnp.float32),
        ],
        compiler_params=pltpu.CompilerParams(use_tc_tiling_on_sc=False),
        name="gcnii_degree_hist",
    )
    return f(dst3, ones_k, zvec)


def _edge_body(src2_ref, dst2_ref, hn_ref, zrows_ref, out_ref,
               idxsb, idxdb, rows0, rows1, acc, semA, semB):
    if True:
        c = lax.axis_index("c")
        s = lax.axis_index("s")
        r0 = s * NODE_PER_TILE
        pltpu.sync_copy(zrows_ref.at[pl.ds(r0, NODE_PER_TILE), :],
                        acc.at[pl.ds(r0, NODE_PER_TILE), :])
        plsc.subcore_barrier()

        row0 = c * CHUNK_PER_SC + s * ROWS_PER_TILE
        for blk in range(ROWS_PER_TILE // IB):
            b0 = row0 + blk * IB
            pltpu.sync_copy(src2_ref.at[pl.ds(b0, IB), :], idxsb)
            pltpu.sync_copy(dst2_ref.at[pl.ds(b0, IB), :], idxdb)

            pltpu.async_copy(hn_ref.at[idxsb.at[0]], rows0, semA)

            def body(i, carry):
                j = 2 * i
                pltpu.async_copy(hn_ref.at[idxsb.at[j + 1]], rows1, semB)
                pltpu.make_async_copy(hn_ref.at[idxsb.at[j]], rows0,
                                      semA).wait()
                pltpu.sync_copy(rows0, acc.at[idxdb.at[j]], add=True)

                @pl.when(j + 2 < IB)
                def _():
                    pltpu.async_copy(hn_ref.at[idxsb.at[j + 2]], rows0, semA)

                pltpu.make_async_copy(hn_ref.at[idxsb.at[j]], rows1,
                                      semB).wait()
                pltpu.sync_copy(rows1, acc.at[idxdb.at[j + 1]], add=True)
                return carry

            lax.fori_loop(0, IB // 2, body, 0)

        @pl.when(s < LEFTOVER)
        def _():
            rowx = c * CHUNK_PER_SC + NS * ROWS_PER_TILE + s
            pltpu.sync_copy(src2_ref.at[rowx, :], idxsb.at[0])
            pltpu.sync_copy(dst2_ref.at[rowx, :], idxdb.at[0])
            pltpu.sync_copy(hn_ref.at[idxsb.at[0]], rows0)
            pltpu.sync_copy(rows0, acc.at[idxdb.at[0]], add=True)

        plsc.subcore_barrier()
        pltpu.sync_copy(acc.at[pl.ds(r0, NODE_PER_TILE), :],
                        out_ref.at[c, pl.ds(r0, NODE_PER_TILE), :])


def _edge_scatter(src2, dst2, hn, zrows):
    f = pl.kernel(
        _edge_body,
        out_type=jax.ShapeDtypeStruct((NC, N, H), jnp.float32),
        mesh=_sc_mesh(),
        scratch_types=[
            pltpu.VMEM((IB, K), jnp.int32),
            pltpu.VMEM((IB, K), jnp.int32),
            pltpu.VMEM((K, H), jnp.float32),
            pltpu.VMEM((K, H), jnp.float32),
            pltpu.VMEM_SHARED((N, H), jnp.float32),
            pltpu.SemaphoreType.DMA,
            pltpu.SemaphoreType.DMA,
        ],
        compiler_params=pltpu.CompilerParams(use_tc_tiling_on_sc=False),
        name="gcnii_edge_scatter",
    )
    return f(src2, dst2, hn, zrows)


def _tc_prep(x, W_in, b_in, cntT):
    def body(x_ref, wi_ref, bi_ref, cnt_ref, h_ref, hn0_ref, rinv_ref,
             dinv_ref):
        cnt = cnt_ref[0] + cnt_ref[1]
        deg = cnt + 1.0
        rinv = lax.rsqrt(deg)
        dinv = 1.0 / deg
        h = jnp.dot(x_ref[...], wi_ref[...],
                    preferred_element_type=jnp.float32) + bi_ref[...][None, :]
        h = jnp.maximum(h, 0.0)
        h_ref[...] = h
        hn0_ref[...] = h * rinv[:, 0:1]
        rinv_ref[...] = rinv
        dinv_ref[...] = dinv

    return pl.pallas_call(
        body,
        out_shape=[
            jax.ShapeDtypeStruct((N, H), jnp.float32),
            jax.ShapeDtypeStruct((N, H), jnp.float32),
            jax.ShapeDtypeStruct((N, L), jnp.float32),
            jax.ShapeDtypeStruct((N, L), jnp.float32),
        ],
        name="gcnii_prep",
    )(x, W_in, b_in, cntT)


def _tc_layer(l, beta, s_part, h, h0, rinv, dinv, W_l, gamma_l, betabn_l):
    def body(sp_ref, h_ref, h0_ref, rinv_ref, dinv_ref, w_ref, g_ref, bb_ref,
             hout_ref, hnout_ref):
        s = sp_ref[0] + sp_ref[1]
        agg = s * rinv_ref[:, l:l + 1] + h_ref[...] * dinv_ref[:, l:l + 1]
        sup = (1.0 - ALPHA) * agg + ALPHA * h0_ref[...]
        t = (1.0 - beta) * sup + beta * jnp.dot(
            sup, w_ref[...], preferred_element_type=jnp.float32)
        mean = jnp.mean(t, axis=0, keepdims=True)
        var = jnp.mean((t - mean) ** 2, axis=0, keepdims=True)
        hb = (t - mean) * lax.rsqrt(var + BN_EPS) * g_ref[...][None, :] \
            + bb_ref[...][None, :]
        hnew = jnp.maximum(hb, 0.0)
        hout_ref[...] = hnew
        hnout_ref[...] = hnew * rinv_ref[:, l + 1:l + 2]

    return pl.pallas_call(
        body,
        out_shape=[
            jax.ShapeDtypeStruct((N, H), jnp.float32),
            jax.ShapeDtypeStruct((N, H), jnp.float32),
        ],
        name=f"gcnii_layer{l}",
    )(s_part, h, h0, rinv, dinv, W_l, gamma_l, betabn_l)


def _tc_last(l, beta, s_part, h, h0, rinv, dinv, W_l, gamma_l, betabn_l,
             W_out, b_out):
    def body(sp_ref, h_ref, h0_ref, rinv_ref, dinv_ref, w_ref, g_ref, bb_ref,
             wo_ref, bo_ref, out_ref):
        s = sp_ref[0] + sp_ref[1]
        agg = s * rinv_ref[:, l:l + 1] + h_ref[...] * dinv_ref[:, l:l + 1]
        sup = (1.0 - ALPHA) * agg + ALPHA * h0_ref[...]
        t = (1.0 - beta) * sup + beta * jnp.dot(
            sup, w_ref[...], preferred_element_type=jnp.float32)
        mean = jnp.mean(t, axis=0, keepdims=True)
        var = jnp.mean((t - mean) ** 2, axis=0, keepdims=True)
        hb = (t - mean) * lax.rsqrt(var + BN_EPS) * g_ref[...][None, :] \
            + bb_ref[...][None, :]
        hnew = jnp.maximum(hb, 0.0)
        out_ref[...] = jnp.dot(hnew, wo_ref[...],
                               preferred_element_type=jnp.float32) \
            + bo_ref[...][None, :]

    return pl.pallas_call(
        body,
        out_shape=jax.ShapeDtypeStruct((N, C), jnp.float32),
        name="gcnii_last",
    )(s_part, h, h0, rinv, dinv, W_l, gamma_l, betabn_l, W_out, b_out)


def kernel(x, W_in, b_in, Ws, gammas, betas_bn, W_out, b_out, nodeblocks):
    dst3 = nodeblocks[:, 1, :].reshape(L, NCHUNK, K)
    ones_k = jnp.ones((K,), jnp.float32)
    zvec = jnp.zeros((N,), jnp.float32)
    zrows = jnp.zeros((N, H), jnp.float32)

    cnt_part = _hist(dst3, ones_k, zvec)
    cntT = jnp.transpose(cnt_part, (0, 2, 1))
    h, hn, rinv, dinv = _tc_prep(x, W_in, b_in, cntT)
    h0 = h
    out = None
    for l in range(L):
        src2 = nodeblocks[l, 0].reshape(NCHUNK, K)
        dst2 = dst3[l]
        s_part = _edge_scatter(src2, dst2, hn, zrows)
        beta = float(math.log(LAMBDA / (l + 1) + 1.0))
        if l < L - 1:
            h, hn = _tc_layer(l, beta, s_part, h, h0, rinv, dinv,
                              Ws[l], gammas[l], betas_bn[l])
        else:
            out = _tc_last(l, beta, s_part, h, h0, rinv, dinv,
                           Ws[l], gammas[l], betas_bn[l], W_out, b_out)
    return out


# speedup vs baseline: 3.2225x; 1.0409x over previous
import math

import jax
import jax.numpy as jnp
from jax import lax
from jax.experimental import pallas as pl
from jax.experimental.pallas import tpu as pltpu
from jax.experimental.pallas import tpu_sc as plsc

N = 10000
E = 320000
L = 4
H = 128
C = 40
ALPHA = 0.1
LAMBDA = 0.5
BN_EPS = 1e-5

NC = 2
NS = 16
K = 128
NCHUNK = E // K
CHUNK_PER_SC = NCHUNK // NC
ROWS_PER_TILE = CHUNK_PER_SC // NS
LEFTOVER = CHUNK_PER_SC - ROWS_PER_TILE * NS
NODE_PER_TILE = N // NS
IB = 26


def _sc_mesh():
    return plsc.VectorSubcoreMesh(core_axis_name="c", subcore_axis_name="s")


def _hist_body(dst3_ref, ones_ref, zvec_ref, out_ref, idxb, ones_v,
               acc0, acc1, acc2, acc3, semH):
    c = lax.axis_index("c")
    s = lax.axis_index("s")
    accs = [acc0, acc1, acc2, acc3]

    @pl.when(s == 0)
    def _():
        for a in accs:
            pltpu.sync_copy(zvec_ref, a)

    pltpu.sync_copy(ones_ref, ones_v)
    plsc.subcore_barrier()

    row0 = c * CHUNK_PER_SC + s * ROWS_PER_TILE
    for l in range(L):
        pltpu.sync_copy(dst3_ref.at[l, pl.ds(row0, ROWS_PER_TILE), :], idxb)

        def body(g, carry, l=l):
            for i in range(8):
                j = 8 * g + i

                @pl.when(j < ROWS_PER_TILE)
                def _(j=j, l=l):
                    pltpu.async_copy(ones_v, accs[l].at[idxb.at[j]], semH,
                                     add=True)
            for i in range(8):
                j = 8 * g + i

                @pl.when(j < ROWS_PER_TILE)
                def _(j=j, l=l):
                    pltpu.make_async_copy(ones_v, accs[l].at[idxb.at[j]],
                                          semH).wait()
            return carry

        lax.fori_loop(0, (ROWS_PER_TILE + 7) // 8, body, 0)

        @pl.when(s < LEFTOVER)
        def _(l=l):
            rowx = c * CHUNK_PER_SC + NS * ROWS_PER_TILE + s
            pltpu.sync_copy(dst3_ref.at[l, rowx, :], idxb.at[0])
            pltpu.sync_copy(ones_v, accs[l].at[idxb.at[0]], add=True)

    plsc.subcore_barrier()

    @pl.when(s == 0)
    def _():
        for l in range(L):
            pltpu.sync_copy(accs[l], out_ref.at[c, l])


def _hist(dst3, ones_k, zvec):
    f = pl.kernel(
        _hist_body,
        out_type=jax.ShapeDtypeStruct((NC, L, N), jnp.float32),
        mesh=_sc_mesh(),
        scratch_types=[
            pltpu.VMEM((ROWS_PER_TILE, K), jnp.int32),
            pltpu.VMEM((K,), jnp.float32),
            pltpu.VMEM_SHARED((N,), jnp.float32),
            pltpu.VMEM_SHARED((N,), jnp.float32),
            pltpu.VMEM_SHARED((N,), jnp.float32),
            pltpu.VMEM_SHARED((N,), jnp.float32),
            pltpu.SemaphoreType.DMA,
        ],
        compiler_params=pltpu.CompilerParams(use_tc_tiling_on_sc=False),
        name="gcnii_degree_hist",
    )
    return f(dst3, ones_k, zvec)


def _edge_body(src2_ref, dst2_ref, hn_ref, zrows_ref, out_ref,
               idxsb, idxdb, idxsb2, idxdb2, rows0, rows1, acc,
               semA, semB, semI):
    if True:
        c = lax.axis_index("c")
        s = lax.axis_index("s")
        r0 = s * NODE_PER_TILE
        pltpu.sync_copy(zrows_ref.at[pl.ds(r0, NODE_PER_TILE), :],
                        acc.at[pl.ds(r0, NODE_PER_TILE), :])
        plsc.subcore_barrier()

        row0 = c * CHUNK_PER_SC + s * ROWS_PER_TILE
        nblk = ROWS_PER_TILE // IB
        pairs = [(idxsb, idxdb), (idxsb2, idxdb2)]
        pltpu.sync_copy(src2_ref.at[pl.ds(row0, IB), :], idxsb)
        pltpu.sync_copy(dst2_ref.at[pl.ds(row0, IB), :], idxdb)
        for blk in range(nblk):
            isb, idb = pairs[blk % 2]
            nsb, ndb = pairs[(blk + 1) % 2]
            if blk + 1 < nblk:
                b1 = row0 + (blk + 1) * IB
                pltpu.async_copy(src2_ref.at[pl.ds(b1, IB), :], nsb, semI)
                pltpu.async_copy(dst2_ref.at[pl.ds(b1, IB), :], ndb, semI)

            pltpu.async_copy(hn_ref.at[isb.at[0]], rows0, semA)

            def body(i, carry, isb=isb, idb=idb):
                j = 2 * i
                pltpu.async_copy(hn_ref.at[isb.at[j + 1]], rows1, semB)
                pltpu.make_async_copy(hn_ref.at[isb.at[j]], rows0,
                                      semA).wait()
                pltpu.sync_copy(rows0, acc.at[idb.at[j]], add=True)

                @pl.when(j + 2 < IB)
                def _():
                    pltpu.async_copy(hn_ref.at[isb.at[j + 2]], rows0, semA)

                pltpu.make_async_copy(hn_ref.at[isb.at[j]], rows1,
                                      semB).wait()
                pltpu.sync_copy(rows1, acc.at[idb.at[j + 1]], add=True)
                return carry

            lax.fori_loop(0, IB // 2, body, 0)

            if blk + 1 < nblk:
                pltpu.make_async_copy(src2_ref.at[pl.ds(row0, IB), :], nsb,
                                      semI).wait()
                pltpu.make_async_copy(dst2_ref.at[pl.ds(row0, IB), :], ndb,
                                      semI).wait()

        @pl.when(s < LEFTOVER)
        def _():
            rowx = c * CHUNK_PER_SC + NS * ROWS_PER_TILE + s
            pltpu.sync_copy(src2_ref.at[rowx, :], idxsb.at[0])
            pltpu.sync_copy(dst2_ref.at[rowx, :], idxdb.at[0])
            pltpu.sync_copy(hn_ref.at[idxsb.at[0]], rows0)
            pltpu.sync_copy(rows0, acc.at[idxdb.at[0]], add=True)

        plsc.subcore_barrier()
        pltpu.sync_copy(acc.at[pl.ds(r0, NODE_PER_TILE), :],
                        out_ref.at[c, pl.ds(r0, NODE_PER_TILE), :])


def _edge_scatter(src2, dst2, hn, zrows):
    f = pl.kernel(
        _edge_body,
        out_type=jax.ShapeDtypeStruct((NC, N, H), jnp.float32),
        mesh=_sc_mesh(),
        scratch_types=[
            pltpu.VMEM((IB, K), jnp.int32),
            pltpu.VMEM((IB, K), jnp.int32),
            pltpu.VMEM((IB, K), jnp.int32),
            pltpu.VMEM((IB, K), jnp.int32),
            pltpu.VMEM((K, H), jnp.float32),
            pltpu.VMEM((K, H), jnp.float32),
            pltpu.VMEM_SHARED((N, H), jnp.float32),
            pltpu.SemaphoreType.DMA,
            pltpu.SemaphoreType.DMA,
            pltpu.SemaphoreType.DMA,
        ],
        compiler_params=pltpu.CompilerParams(use_tc_tiling_on_sc=False),
        name="gcnii_edge_scatter",
    )
    return f(src2, dst2, hn, zrows)


def _tc_prep(x, W_in, b_in, cntT):
    def body(x_ref, wi_ref, bi_ref, cnt_ref, h_ref, hn0_ref, rinv_ref,
             dinv_ref):
        cnt = cnt_ref[0] + cnt_ref[1]
        deg = cnt + 1.0
        rinv = lax.rsqrt(deg)
        dinv = 1.0 / deg
        h = jnp.dot(x_ref[...], wi_ref[...],
                    preferred_element_type=jnp.float32) + bi_ref[...][None, :]
        h = jnp.maximum(h, 0.0)
        h_ref[...] = h
        hn0_ref[...] = h * rinv[:, 0:1]
        rinv_ref[...] = rinv
        dinv_ref[...] = dinv

    return pl.pallas_call(
        body,
        out_shape=[
            jax.ShapeDtypeStruct((N, H), jnp.float32),
            jax.ShapeDtypeStruct((N, H), jnp.float32),
            jax.ShapeDtypeStruct((N, L), jnp.float32),
            jax.ShapeDtypeStruct((N, L), jnp.float32),
        ],
        name="gcnii_prep",
    )(x, W_in, b_in, cntT)


def _tc_layer(l, beta, s_part, h, h0, rinv, dinv, W_l, gamma_l, betabn_l):
    def body(sp_ref, h_ref, h0_ref, rinv_ref, dinv_ref, w_ref, g_ref, bb_ref,
             hout_ref, hnout_ref):
        s = sp_ref[0] + sp_ref[1]
        agg = s * rinv_ref[:, l:l + 1] + h_ref[...] * dinv_ref[:, l:l + 1]
        sup = (1.0 - ALPHA) * agg + ALPHA * h0_ref[...]
        t = (1.0 - beta) * sup + beta * jnp.dot(
            sup, w_ref[...], preferred_element_type=jnp.float32)
        mean = jnp.mean(t, axis=0, keepdims=True)
        var = jnp.mean((t - mean) ** 2, axis=0, keepdims=True)
        hb = (t - mean) * lax.rsqrt(var + BN_EPS) * g_ref[...][None, :] \
            + bb_ref[...][None, :]
        hnew = jnp.maximum(hb, 0.0)
        hout_ref[...] = hnew
        hnout_ref[...] = hnew * rinv_ref[:, l + 1:l + 2]

    return pl.pallas_call(
        body,
        out_shape=[
            jax.ShapeDtypeStruct((N, H), jnp.float32),
            jax.ShapeDtypeStruct((N, H), jnp.float32),
        ],
        name=f"gcnii_layer{l}",
    )(s_part, h, h0, rinv, dinv, W_l, gamma_l, betabn_l)


def _tc_last(l, beta, s_part, h, h0, rinv, dinv, W_l, gamma_l, betabn_l,
             W_out, b_out):
    def body(sp_ref, h_ref, h0_ref, rinv_ref, dinv_ref, w_ref, g_ref, bb_ref,
             wo_ref, bo_ref, out_ref):
        s = sp_ref[0] + sp_ref[1]
        agg = s * rinv_ref[:, l:l + 1] + h_ref[...] * dinv_ref[:, l:l + 1]
        sup = (1.0 - ALPHA) * agg + ALPHA * h0_ref[...]
        t = (1.0 - beta) * sup + beta * jnp.dot(
            sup, w_ref[...], preferred_element_type=jnp.float32)
        mean = jnp.mean(t, axis=0, keepdims=True)
        var = jnp.mean((t - mean) ** 2, axis=0, keepdims=True)
        hb = (t - mean) * lax.rsqrt(var + BN_EPS) * g_ref[...][None, :] \
            + bb_ref[...][None, :]
        hnew = jnp.maximum(hb, 0.0)
        out_ref[...] = jnp.dot(hnew, wo_ref[...],
                               preferred_element_type=jnp.float32) \
            + bo_ref[...][None, :]

    return pl.pallas_call(
        body,
        out_shape=jax.ShapeDtypeStruct((N, C), jnp.float32),
        name="gcnii_last",
    )(s_part, h, h0, rinv, dinv, W_l, gamma_l, betabn_l, W_out, b_out)


def kernel(x, W_in, b_in, Ws, gammas, betas_bn, W_out, b_out, nodeblocks):
    dst3 = nodeblocks[:, 1, :].reshape(L, NCHUNK, K)
    ones_k = jnp.ones((K,), jnp.float32)
    zvec = jnp.zeros((N,), jnp.float32)
    zrows = jnp.zeros((N, H), jnp.float32)

    cnt_part = _hist(dst3, ones_k, zvec)
    cntT = jnp.transpose(cnt_part, (0, 2, 1))
    h, hn, rinv, dinv = _tc_prep(x, W_in, b_in, cntT)
    h0 = h
    out = None
    for l in range(L):
        src2 = nodeblocks[l, 0].reshape(NCHUNK, K)
        dst2 = dst3[l]
        s_part = _edge_scatter(src2, dst2, hn, zrows)
        beta = float(math.log(LAMBDA / (l + 1) + 1.0))
        if l < L - 1:
            h, hn = _tc_layer(l, beta, s_part, h, h0, rinv, dinv,
                              Ws[l], gammas[l], betas_bn[l])
        else:
            out = _tc_last(l, beta, s_part, h, h0, rinv, dinv,
                           Ws[l], gammas[l], betas_bn[l], W_out, b_out)
    return out
